# baseline (speedup 1.0000x reference)
import functools

import jax
import jax.numpy as jnp
from jax import lax
from jax.experimental import pallas as pl
from jax.experimental.pallas import tpu as pltpu

W = 1024
EPS = 1e-10
INVALID = -1


def _k1_body(ids_ref, d_ref, t_ref, sr_ref, st_ref, q_ref, p_ref, *, V, C):
    B, Kp1, _ = t_ref.shape
    K = Kp1 - 1
    c = pl.program_id(0)
    d = d_ref[...]
    t = t_ref[...]
    li = lax.broadcasted_iota(jnp.int32, (B, K, W), 2) + c * W

    @pl.when(c < C - 1)
    def _():
        r = jnp.maximum(t[:, :K, :] - d, 0.0)
        sr_ref[0] = r.sum(-1)
        st_ref[0] = t.sum(-1)

    @pl.when(c == C - 1)
    def _():
        lit = lax.broadcasted_iota(jnp.int32, (B, Kp1, W), 2) + c * W
        tm = jnp.where(lit < V, t, 0.0)
        dm = jnp.where(li < V, d, 0.0)
        r = jnp.maximum(tm[:, :K, :] - dm, 0.0)
        sr_ref[0] = r.sum(-1)
        st_ref[0] = tm.sum(-1)

    ids = ids_ref[...]
    m = li == ids[:, :, None]
    qp = jnp.where(m, d, 0.0).sum(-1)
    pp = jnp.where(m, t[:, :K, :], 0.0).sum(-1)

    @pl.when(c == 0)
    def _():
        q_ref[...] = jnp.zeros_like(q_ref)
        p_ref[...] = jnp.zeros_like(p_ref)

    q_ref[...] += qp
    p_ref[...] += pp


def _run_k1(draft, target, ids):
    B, K, V = draft.shape
    C = pl.cdiv(V, W)
    return pl.pallas_call(
        functools.partial(_k1_body, V=V, C=C),
        grid=(C,),
        in_specs=[
            pl.BlockSpec((B, K), lambda c: (0, 0)),
            pl.BlockSpec((B, K, W), lambda c: (0, 0, c)),
            pl.BlockSpec((B, K + 1, W), lambda c: (0, 0, c)),
        ],
        out_specs=[
            pl.BlockSpec((1, B, K), lambda c: (c, 0, 0)),
            pl.BlockSpec((1, B, K + 1), lambda c: (c, 0, 0)),
            pl.BlockSpec((B, K), lambda c: (0, 0)),
            pl.BlockSpec((B, K), lambda c: (0, 0)),
        ],
        out_shape=[
            jax.ShapeDtypeStruct((C, B, K), jnp.float32),
            jax.ShapeDtypeStruct((C, B, K + 1), jnp.float32),
            jax.ShapeDtypeStruct((B, K), jnp.float32),
            jax.ShapeDtypeStruct((B, K), jnp.float32),
        ],
    )(ids, draft, target)


def _k2_body(sr_ref, st_ref, u_ref, q_ref, p_ref, ua_ref,
             cstar_ref, base_ref, thr_ref, flag_ref, na_ref):
    C, N = sr_ref.shape
    sr = sr_ref[...]
    st = st_ref[...]
    u = u_ref[...]

    s_tot = sr.sum(0, keepdims=True)
    kpos = lax.broadcasted_iota(jnp.int32, (1, N), 1) % 9
    use_r = (s_tot > EPS) & (kpos < 8)
    sel = jnp.where(use_r, sr, st)
    thr = jnp.where(use_r, u * s_tot, u)

    li_ = lax.broadcasted_iota(jnp.int32, (C, C), 0)
    lj_ = lax.broadcasted_iota(jnp.int32, (C, C), 1)
    ltri = (lj_ <= li_).astype(jnp.float32)
    cc = jax.lax.dot_general(ltri, sel, (((1,), (0,)), ((), ())),
                             preferred_element_type=jnp.float32)
    cstar = (cc < thr).astype(jnp.int32).sum(0, keepdims=True)
    cstar = jnp.minimum(cstar, C - 1)
    ci = lax.broadcasted_iota(jnp.int32, (C, N), 0)
    base = jnp.where(ci < cstar, sel, 0.0).sum(0, keepdims=True)

    cstar_ref[...] = cstar
    base_ref[...] = base
    thr_ref[...] = thr
    flag_ref[...] = use_r.astype(jnp.int32)

    q = q_ref[...]
    p = p_ref[...]
    ua = ua_ref[...]
    acc_prob = jnp.minimum(1.0, p / jnp.maximum(q, EPS))
    rejected = (ua > acc_prob).astype(jnp.float32)
    K = rejected.shape[1]
    ki_ = lax.broadcasted_iota(jnp.int32, (K, K), 0)
    kj_ = lax.broadcasted_iota(jnp.int32, (K, K), 1)
    utri = (ki_ <= kj_).astype(jnp.float32)
    cumrej = jax.lax.dot_general(rejected, utri, (((1,), (0,)), ((), ())),
                                 preferred_element_type=jnp.float32)
    na = (cumrej == 0.0).astype(jnp.int32).sum(-1, keepdims=True)
    na_ref[...] = na


def _run_k2(sr_pad, st, u_flat, q, p, ua):
    B = q.shape[0]
    N = sr_pad.shape[1]
    return pl.pallas_call(
        _k2_body,
        out_shape=[
            jax.ShapeDtypeStruct((1, N), jnp.int32),
            jax.ShapeDtypeStruct((1, N), jnp.float32),
            jax.ShapeDtypeStruct((1, N), jnp.float32),
            jax.ShapeDtypeStruct((1, N), jnp.int32),
            jax.ShapeDtypeStruct((B, 1), jnp.int32),
        ],
    )(sr_pad, st, u_flat, q, p, ua)


def _k3_body(cs_ref, *refs, V):
    b = pl.program_id(0)
    d_ref, t_ref = refs[0], refs[1]
    thr_ref, base_ref, flag_ref, na_ref, ids_ref = refs[2:7]
    out_ref = refs[7]

    thr = thr_ref[0]
    base = base_ref[0]
    flag = flag_ref[0]

    rows = []
    cbase = []
    for k in range(9):
        cstar_k = cs_ref[b * 9 + k]
        t = t_ref[k][0:1, k:k + 1, :][0]
        if k < 8:
            d = d_ref[k][0:1, k:k + 1, :][0]
            fk = flag[0:1, k:k + 1]
            vals = jnp.where(fk > 0, jnp.maximum(t - d, 0.0), t)
        else:
            vals = t
        li = lax.broadcasted_iota(jnp.int32, (1, W), 1) + cstar_k * W
        vals = jnp.where(li < V, vals, 0.0)
        rows.append(vals)
        cbase.append(cstar_k * W)
    vals9 = jnp.concatenate(rows, axis=0)

    wi_ = lax.broadcasted_iota(jnp.int32, (W, W), 0)
    wj_ = lax.broadcasted_iota(jnp.int32, (W, W), 1)
    utri = (wi_ <= wj_).astype(jnp.float32)
    cum9 = jax.lax.dot_general(vals9, utri, (((1,), (0,)), ((), ())),
                               preferred_element_type=jnp.float32)
    toks = []
    for k in range(9):
        cum_k = cum9[k:k + 1, :] + base[0:1, k:k + 1]
        cnt_k = (cum_k < thr[0:1, k:k + 1]).astype(jnp.int32).sum(
            -1, keepdims=True)
        toks.append(jnp.minimum(cbase[k] + cnt_k, V - 1))
    rec = jnp.concatenate(toks, axis=-1)

    ids_ext = jnp.concatenate(
        [ids_ref[0], jnp.zeros((1, 1), jnp.int32)], axis=-1)
    pos = lax.broadcasted_iota(jnp.int32, (1, 9), 1)
    na = na_ref[0]
    out_ref[0] = jnp.where(pos < na, ids_ext,
                           jnp.where(pos == na, rec,
                                     jnp.full((1, 9), INVALID, jnp.int32)))


def _run_k3(cstar_flat, draft, target, thr, base, flag, na, ids):
    B, K, V = draft.shape
    d_specs = [
        pl.BlockSpec((1, K, W), functools.partial(
            lambda b, cs, kk: (b, 0, cs[b * 9 + kk]), kk=k))
        for k in range(8)
    ]
    t_specs = [
        pl.BlockSpec((1, K + 1, W), functools.partial(
            lambda b, cs, kk: (b, 0, cs[b * 9 + kk]), kk=k))
        for k in range(9)
    ]
    grid_spec = pltpu.PrefetchScalarGridSpec(
        num_scalar_prefetch=1,
        grid=(B,),
        in_specs=d_specs + t_specs + [
            pl.BlockSpec((1, 1, 9), lambda b, cs: (b, 0, 0)),
            pl.BlockSpec((1, 1, 9), lambda b, cs: (b, 0, 0)),
            pl.BlockSpec((1, 1, 9), lambda b, cs: (b, 0, 0)),
            pl.BlockSpec((1, 1, 1), lambda b, cs: (b, 0, 0)),
            pl.BlockSpec((1, 1, 8), lambda b, cs: (b, 0, 0)),
        ],
        out_specs=pl.BlockSpec((1, 1, 9), lambda b, cs: (b, 0, 0)),
    )

    def body(cs_ref, *refs):
        d_refs = refs[0:8]
        t_refs = refs[8:17]
        rest = refs[17:]
        return _k3_body(cs_ref, d_refs, t_refs, *rest, V=V)

    out = pl.pallas_call(
        body,
        grid_spec=grid_spec,
        out_shape=jax.ShapeDtypeStruct((B, 1, 9), jnp.int32),
    )(cstar_flat, *([draft] * 8), *([target] * 9),
      thr, base, flag, na, ids)
    return out.reshape(B, 9)


def kernel(draft_probs, target_probs, uniform_accept, uniform_sample,
           draft_token_ids):
    B, K, V = draft_probs.shape
    C = pl.cdiv(V, W)
    srT, stT, q, p = _run_k1(draft_probs, target_probs, draft_token_ids)

    sr_pad = jnp.concatenate(
        [srT, jnp.zeros((C, B, 1), jnp.float32)], axis=-1).reshape(C, B * (K + 1))
    st_flat = stT.reshape(C, B * (K + 1))
    u_flat = uniform_sample.reshape(1, B * (K + 1))

    cstar, base, thr, flag, na = _run_k2(sr_pad, st_flat, u_flat, q, p,
                                         uniform_accept)

    thr9 = thr.reshape(B, 1, K + 1)
    base9 = base.reshape(B, 1, K + 1)
    flag9 = flag.reshape(B, 1, K + 1)
    na9 = na.reshape(B, 1, 1)
    ids9 = draft_token_ids.reshape(B, 1, K)
    cstar_flat = cstar.reshape(B * (K + 1))

    return _run_k3(cstar_flat, draft_probs, target_probs,
                   thr9, base9, flag9, na9, ids9)


# speedup vs baseline: 1.1994x; 1.1994x over previous
import functools

import jax
import jax.numpy as jnp
from jax import lax
from jax.experimental import pallas as pl
from jax.experimental.pallas import tpu as pltpu

W = 1024
EPS = 1e-10
INVALID = -1


def _k1_body(ids_ref, d_ref, t_ref, sr_ref, st_ref, q_ref, p_ref, *, V, C):
    _, Kp1, WB = t_ref.shape
    K = Kp1 - 1
    d = d_ref[0]
    t = t_ref[0]
    t8 = t[:K, :]
    r = jnp.maximum(t8 - d, 0.0)

    for c in range(C):
        rs = r[:, c * W:(c + 1) * W]
        ts = t[:, c * W:(c + 1) * W]
        if (c + 1) * W > V:
            m = (lax.broadcasted_iota(jnp.int32, (Kp1, W), 1) + c * W) < V
            ts = jnp.where(m, ts, 0.0)
            rs = jnp.where(m[:K], rs, 0.0)
        sr_ref[0, :, c:c + 1] = rs.sum(-1, keepdims=True)
        st_ref[0, :, c:c + 1] = ts.sum(-1, keepdims=True)

    ids = ids_ref[0]
    li = lax.broadcasted_iota(jnp.int32, (K, WB), 1)
    m = li == ids
    q_ref[0] = jnp.where(m, d, 0.0).sum(-1, keepdims=True)
    p_ref[0] = jnp.where(m, t8, 0.0).sum(-1, keepdims=True)


def _run_k1(draft, target, ids):
    B, K, V = draft.shape
    C = pl.cdiv(V, W)
    WB = C * W
    return pl.pallas_call(
        functools.partial(_k1_body, V=V, C=C),
        grid=(B,),
        in_specs=[
            pl.BlockSpec((1, K, 1), lambda b: (b, 0, 0)),
            pl.BlockSpec((1, K, WB), lambda b: (b, 0, 0)),
            pl.BlockSpec((1, K + 1, WB), lambda b: (b, 0, 0)),
        ],
        out_specs=[
            pl.BlockSpec((1, K, C), lambda b: (b, 0, 0)),
            pl.BlockSpec((1, K + 1, C), lambda b: (b, 0, 0)),
            pl.BlockSpec((1, K, 1), lambda b: (b, 0, 0)),
            pl.BlockSpec((1, K, 1), lambda b: (b, 0, 0)),
        ],
        out_shape=[
            jax.ShapeDtypeStruct((B, K, C), jnp.float32),
            jax.ShapeDtypeStruct((B, K + 1, C), jnp.float32),
            jax.ShapeDtypeStruct((B, K, 1), jnp.float32),
            jax.ShapeDtypeStruct((B, K, 1), jnp.float32),
        ],
    )(ids.reshape(B, K, 1), draft, target)


def _k2_body(sr_ref, st_ref, u_ref, q_ref, p_ref, ua_ref,
             cstar_ref, base_ref, thr_ref, flag_ref, na_ref):
    N, C = sr_ref.shape
    sr = sr_ref[...]
    st = st_ref[...]
    u = u_ref[...]

    s_tot = sr.sum(-1, keepdims=True)
    kpos = lax.broadcasted_iota(jnp.int32, (N, 1), 0) % 9
    use_r = (s_tot > EPS) & (kpos < 8)
    sel = jnp.where(use_r, sr, st)
    thr = jnp.where(use_r, u * s_tot, u)

    ci_ = lax.broadcasted_iota(jnp.int32, (C, C), 0)
    cj_ = lax.broadcasted_iota(jnp.int32, (C, C), 1)
    utri = (ci_ <= cj_).astype(jnp.float32)
    cc = jax.lax.dot_general(sel, utri, (((1,), (0,)), ((), ())),
                             preferred_element_type=jnp.float32)
    cstar = (cc < thr).astype(jnp.int32).sum(-1, keepdims=True)
    cstar = jnp.minimum(cstar, C - 1)
    cj = lax.broadcasted_iota(jnp.int32, (N, C), 1)
    base = jnp.where(cj < cstar, sel, 0.0).sum(-1, keepdims=True)

    cstar_ref[...] = cstar
    base_ref[...] = base
    thr_ref[...] = thr
    flag_ref[...] = use_r.astype(jnp.int32)

    q = q_ref[...]
    p = p_ref[...]
    ua = ua_ref[...]
    acc_prob = jnp.minimum(1.0, p / jnp.maximum(q, EPS))
    rejected = (ua > acc_prob).astype(jnp.float32)
    K = rejected.shape[1]
    ki_ = lax.broadcasted_iota(jnp.int32, (K, K), 0)
    kj_ = lax.broadcasted_iota(jnp.int32, (K, K), 1)
    ktri = (ki_ <= kj_).astype(jnp.float32)
    cumrej = jax.lax.dot_general(rejected, ktri, (((1,), (0,)), ((), ())),
                                 preferred_element_type=jnp.float32)
    na = (cumrej == 0.0).astype(jnp.int32).sum(-1, keepdims=True)
    na_ref[...] = na


def _run_k2(sr_pairs, st_pairs, u_col, q, p, ua):
    B = q.shape[0]
    N = sr_pairs.shape[0]
    return pl.pallas_call(
        _k2_body,
        out_shape=[
            jax.ShapeDtypeStruct((N, 1), jnp.int32),
            jax.ShapeDtypeStruct((N, 1), jnp.float32),
            jax.ShapeDtypeStruct((N, 1), jnp.float32),
            jax.ShapeDtypeStruct((N, 1), jnp.int32),
            jax.ShapeDtypeStruct((B, 1), jnp.int32),
        ],
    )(sr_pairs, st_pairs, u_col, q, p, ua)


def _k3_body(cs_ref, d_ref, t_ref, thr_ref, base_ref, flag_ref, na_ref,
             ids_ref, out_ref, *, V):
    b = pl.program_id(0)

    thr = thr_ref[0]
    base = base_ref[0]
    flag = flag_ref[0]

    rows = []
    cbase = []
    for k in range(9):
        cstar_k = cs_ref[b * 9 + k]
        t = t_ref[k][0:1, k:k + 1, :][0]
        if k < 8:
            d = d_ref[k][0:1, k:k + 1, :][0]
            fk = flag[0:1, k:k + 1]
            vals = jnp.where(fk > 0, jnp.maximum(t - d, 0.0), t)
        else:
            vals = t
        li = lax.broadcasted_iota(jnp.int32, (1, W), 1) + cstar_k * W
        vals = jnp.where(li < V, vals, 0.0)
        rows.append(vals)
        cbase.append(cstar_k * W)
    vals9 = jnp.concatenate(rows, axis=0)

    wi_ = lax.broadcasted_iota(jnp.int32, (W, W), 0)
    wj_ = lax.broadcasted_iota(jnp.int32, (W, W), 1)
    utri = (wi_ <= wj_).astype(jnp.float32)
    cum9 = jax.lax.dot_general(vals9, utri, (((1,), (0,)), ((), ())),
                               preferred_element_type=jnp.float32)
    toks = []
    for k in range(9):
        cum_k = cum9[k:k + 1, :] + base[0:1, k:k + 1]
        cnt_k = (cum_k < thr[0:1, k:k + 1]).astype(jnp.int32).sum(
            -1, keepdims=True)
        toks.append(jnp.minimum(cbase[k] + cnt_k, V - 1))
    rec = jnp.concatenate(toks, axis=-1)

    ids_ext = jnp.concatenate(
        [ids_ref[0], jnp.zeros((1, 1), jnp.int32)], axis=-1)
    pos = lax.broadcasted_iota(jnp.int32, (1, 9), 1)
    na = na_ref[0]
    out_ref[0] = jnp.where(pos < na, ids_ext,
                           jnp.where(pos == na, rec,
                                     jnp.full((1, 9), INVALID, jnp.int32)))


def _run_k3(cstar_flat, draft, target, thr, base, flag, na, ids):
    B, K, V = draft.shape
    d_specs = [
        pl.BlockSpec((1, K, W), functools.partial(
            lambda b, cs, kk: (b, 0, cs[b * 9 + kk]), kk=k))
        for k in range(8)
    ]
    t_specs = [
        pl.BlockSpec((1, K + 1, W), functools.partial(
            lambda b, cs, kk: (b, 0, cs[b * 9 + kk]), kk=k))
        for k in range(9)
    ]
    grid_spec = pltpu.PrefetchScalarGridSpec(
        num_scalar_prefetch=1,
        grid=(B,),
        in_specs=d_specs + t_specs + [
            pl.BlockSpec((1, 1, 9), lambda b, cs: (b, 0, 0)),
            pl.BlockSpec((1, 1, 9), lambda b, cs: (b, 0, 0)),
            pl.BlockSpec((1, 1, 9), lambda b, cs: (b, 0, 0)),
            pl.BlockSpec((1, 1, 1), lambda b, cs: (b, 0, 0)),
            pl.BlockSpec((1, 1, 8), lambda b, cs: (b, 0, 0)),
        ],
        out_specs=pl.BlockSpec((1, 1, 9), lambda b, cs: (b, 0, 0)),
    )

    def body(cs_ref, *refs):
        return _k3_body(cs_ref, refs[0:8], refs[8:17], *refs[17:], V=V)

    out = pl.pallas_call(
        body,
        grid_spec=grid_spec,
        out_shape=jax.ShapeDtypeStruct((B, 1, 9), jnp.int32),
    )(cstar_flat, *([draft] * 8), *([target] * 9),
      thr, base, flag, na, ids)
    return out.reshape(B, 9)


def kernel(draft_probs, target_probs, uniform_accept, uniform_sample,
           draft_token_ids):
    B, K, V = draft_probs.shape
    C = pl.cdiv(V, W)
    srT, stT, q3, p3 = _run_k1(draft_probs, target_probs, draft_token_ids)
    q = q3.reshape(B, K)
    p = p3.reshape(B, K)

    sr_pairs = jnp.concatenate(
        [srT, jnp.zeros((B, 1, C), jnp.float32)], axis=1).reshape(B * (K + 1), C)
    st_pairs = stT.reshape(B * (K + 1), C)
    u_col = uniform_sample.reshape(B * (K + 1), 1)

    cstar, base, thr, flag, na = _run_k2(sr_pairs, st_pairs, u_col, q, p,
                                         uniform_accept)

    thr9 = thr.reshape(B, 1, K + 1)
    base9 = base.reshape(B, 1, K + 1)
    flag9 = flag.reshape(B, 1, K + 1)
    na9 = na.reshape(B, 1, 1)
    ids9 = draft_token_ids.reshape(B, 1, K)
    cstar_flat = cstar.reshape(B * (K + 1))

    return _run_k3(cstar_flat, draft_probs, target_probs,
                   thr9, base9, flag9, na9, ids9)


# speedup vs baseline: 1.2013x; 1.0015x over previous
import functools

import jax
import jax.numpy as jnp
from jax import lax
from jax.experimental import pallas as pl
from jax.experimental.pallas import tpu as pltpu

W = 1024
EPS = 1e-10
INVALID = -1


def _k1_body(ids_ref, *refs, V, C, S, CP):
    sr_ref, st_ref, q_ref, p_ref = refs[2 * S:]
    Kp1 = refs[S].shape[1]
    K = Kp1 - 1
    PV = CP * W
    ids = ids_ref[0]

    qacc = jnp.zeros((K, 1), jnp.float32)
    pacc = jnp.zeros((K, 1), jnp.float32)
    for s in range(S):
        d = refs[s][0]
        t = refs[S + s][0]
        t8 = t[:K, :]
        r = jnp.maximum(t8 - d, 0.0)
        for cl in range(CP):
            c = s * CP + cl
            rs = r[:, cl * W:(cl + 1) * W]
            ts = t[:, cl * W:(cl + 1) * W]
            if (c + 1) * W > V:
                m = (lax.broadcasted_iota(jnp.int32, (Kp1, W), 1)
                     + c * W) < V
                ts = jnp.where(m, ts, 0.0)
                rs = jnp.where(m[:K], rs, 0.0)
            sr_ref[0, :, c:c + 1] = rs.sum(-1, keepdims=True)
            st_ref[0, :, c:c + 1] = ts.sum(-1, keepdims=True)
        li = lax.broadcasted_iota(jnp.int32, (K, PV), 1) + s * PV
        m = li == ids
        qacc = qacc + jnp.where(m, d, 0.0).sum(-1, keepdims=True)
        pacc = pacc + jnp.where(m, t8, 0.0).sum(-1, keepdims=True)
    q_ref[0] = qacc
    p_ref[0] = pacc


def _run_k1(draft, target, ids):
    B, K, V = draft.shape
    C = pl.cdiv(V, W)
    S = 7
    CP = C // S
    PV = CP * W
    d_specs = [pl.BlockSpec((1, K, PV), functools.partial(
        lambda b, ss: (b, 0, ss), ss=s)) for s in range(S)]
    t_specs = [pl.BlockSpec((1, K + 1, PV), functools.partial(
        lambda b, ss: (b, 0, ss), ss=s)) for s in range(S)]
    return pl.pallas_call(
        functools.partial(_k1_body, V=V, C=C, S=S, CP=CP),
        grid=(B,),
        in_specs=[pl.BlockSpec((1, K, 1), lambda b: (b, 0, 0))]
        + d_specs + t_specs,
        out_specs=[
            pl.BlockSpec((1, K, C), lambda b: (b, 0, 0)),
            pl.BlockSpec((1, K + 1, C), lambda b: (b, 0, 0)),
            pl.BlockSpec((1, K, 1), lambda b: (b, 0, 0)),
            pl.BlockSpec((1, K, 1), lambda b: (b, 0, 0)),
        ],
        out_shape=[
            jax.ShapeDtypeStruct((B, K, C), jnp.float32),
            jax.ShapeDtypeStruct((B, K + 1, C), jnp.float32),
            jax.ShapeDtypeStruct((B, K, 1), jnp.float32),
            jax.ShapeDtypeStruct((B, K, 1), jnp.float32),
        ],
    )(ids.reshape(B, K, 1), *([draft] * S), *([target] * S))


def _k2_body(sr_ref, st_ref, u_ref, q_ref, p_ref, ua_ref,
             cstar_ref, base_ref, thr_ref, flag_ref, na_ref):
    N, C = sr_ref.shape
    sr = sr_ref[...]
    st = st_ref[...]
    u = u_ref[...]

    s_tot = sr.sum(-1, keepdims=True)
    kpos = lax.broadcasted_iota(jnp.int32, (N, 1), 0) % 9
    use_r = (s_tot > EPS) & (kpos < 8)
    sel = jnp.where(use_r, sr, st)
    thr = jnp.where(use_r, u * s_tot, u)

    ci_ = lax.broadcasted_iota(jnp.int32, (C, C), 0)
    cj_ = lax.broadcasted_iota(jnp.int32, (C, C), 1)
    utri = (ci_ <= cj_).astype(jnp.float32)
    cc = jax.lax.dot_general(sel, utri, (((1,), (0,)), ((), ())),
                             preferred_element_type=jnp.float32)
    cstar = (cc < thr).astype(jnp.int32).sum(-1, keepdims=True)
    cstar = jnp.minimum(cstar, C - 1)
    cj = lax.broadcasted_iota(jnp.int32, (N, C), 1)
    base = jnp.where(cj < cstar, sel, 0.0).sum(-1, keepdims=True)

    cstar_ref[...] = cstar
    base_ref[...] = base
    thr_ref[...] = thr
    flag_ref[...] = use_r.astype(jnp.int32)

    q = q_ref[...]
    p = p_ref[...]
    ua = ua_ref[...]
    acc_prob = jnp.minimum(1.0, p / jnp.maximum(q, EPS))
    rejected = (ua > acc_prob).astype(jnp.float32)
    K = rejected.shape[1]
    ki_ = lax.broadcasted_iota(jnp.int32, (K, K), 0)
    kj_ = lax.broadcasted_iota(jnp.int32, (K, K), 1)
    ktri = (ki_ <= kj_).astype(jnp.float32)
    cumrej = jax.lax.dot_general(rejected, ktri, (((1,), (0,)), ((), ())),
                                 preferred_element_type=jnp.float32)
    na = (cumrej == 0.0).astype(jnp.int32).sum(-1, keepdims=True)
    na_ref[...] = na


def _run_k2(sr_pairs, st_pairs, u_col, q, p, ua):
    B = q.shape[0]
    N = sr_pairs.shape[0]
    return pl.pallas_call(
        _k2_body,
        out_shape=[
            jax.ShapeDtypeStruct((N, 1), jnp.int32),
            jax.ShapeDtypeStruct((N, 1), jnp.float32),
            jax.ShapeDtypeStruct((N, 1), jnp.float32),
            jax.ShapeDtypeStruct((N, 1), jnp.int32),
            jax.ShapeDtypeStruct((B, 1), jnp.int32),
        ],
    )(sr_pairs, st_pairs, u_col, q, p, ua)


def _k3_body(cs_ref, d_ref, t_ref, thr_ref, base_ref, flag_ref, na_ref,
             ids_ref, out_ref, *, V):
    b = pl.program_id(0)

    thr = thr_ref[0]
    base = base_ref[0]
    flag = flag_ref[0]

    rows = []
    cbase = []
    for k in range(9):
        cstar_k = cs_ref[b * 9 + k]
        t = t_ref[k][0:1, k:k + 1, :][0]
        if k < 8:
            d = d_ref[k][0:1, k:k + 1, :][0]
            fk = flag[0:1, k:k + 1]
            vals = jnp.where(fk > 0, jnp.maximum(t - d, 0.0), t)
        else:
            vals = t
        li = lax.broadcasted_iota(jnp.int32, (1, W), 1) + cstar_k * W
        vals = jnp.where(li < V, vals, 0.0)
        rows.append(vals)
        cbase.append(cstar_k * W)
    vals9 = jnp.concatenate(rows, axis=0)

    wi_ = lax.broadcasted_iota(jnp.int32, (W, W), 0)
    wj_ = lax.broadcasted_iota(jnp.int32, (W, W), 1)
    utri = (wi_ <= wj_).astype(jnp.float32)
    cum9 = jax.lax.dot_general(vals9, utri, (((1,), (0,)), ((), ())),
                               preferred_element_type=jnp.float32)
    toks = []
    for k in range(9):
        cum_k = cum9[k:k + 1, :] + base[0:1, k:k + 1]
        cnt_k = (cum_k < thr[0:1, k:k + 1]).astype(jnp.int32).sum(
            -1, keepdims=True)
        toks.append(jnp.minimum(cbase[k] + cnt_k, V - 1))
    rec = jnp.concatenate(toks, axis=-1)

    ids_ext = jnp.concatenate(
        [ids_ref[0], jnp.zeros((1, 1), jnp.int32)], axis=-1)
    pos = lax.broadcasted_iota(jnp.int32, (1, 9), 1)
    na = na_ref[0]
    out_ref[0] = jnp.where(pos < na, ids_ext,
                           jnp.where(pos == na, rec,
                                     jnp.full((1, 9), INVALID, jnp.int32)))


def _run_k3(cstar_flat, draft, target, thr, base, flag, na, ids):
    B, K, V = draft.shape
    d_specs = [
        pl.BlockSpec((1, K, W), functools.partial(
            lambda b, cs, kk: (b, 0, cs[b * 9 + kk]), kk=k))
        for k in range(8)
    ]
    t_specs = [
        pl.BlockSpec((1, K + 1, W), functools.partial(
            lambda b, cs, kk: (b, 0, cs[b * 9 + kk]), kk=k))
        for k in range(9)
    ]
    grid_spec = pltpu.PrefetchScalarGridSpec(
        num_scalar_prefetch=1,
        grid=(B,),
        in_specs=d_specs + t_specs + [
            pl.BlockSpec((1, 1, 9), lambda b, cs: (b, 0, 0)),
            pl.BlockSpec((1, 1, 9), lambda b, cs: (b, 0, 0)),
            pl.BlockSpec((1, 1, 9), lambda b, cs: (b, 0, 0)),
            pl.BlockSpec((1, 1, 1), lambda b, cs: (b, 0, 0)),
            pl.BlockSpec((1, 1, 8), lambda b, cs: (b, 0, 0)),
        ],
        out_specs=pl.BlockSpec((1, 1, 9), lambda b, cs: (b, 0, 0)),
    )

    def body(cs_ref, *refs):
        return _k3_body(cs_ref, refs[0:8], refs[8:17], *refs[17:], V=V)

    out = pl.pallas_call(
        body,
        grid_spec=grid_spec,
        out_shape=jax.ShapeDtypeStruct((B, 1, 9), jnp.int32),
    )(cstar_flat, *([draft] * 8), *([target] * 9),
      thr, base, flag, na, ids)
    return out.reshape(B, 9)


def kernel(draft_probs, target_probs, uniform_accept, uniform_sample,
           draft_token_ids):
    B, K, V = draft_probs.shape
    C = pl.cdiv(V, W)
    srT, stT, q3, p3 = _run_k1(draft_probs, target_probs, draft_token_ids)
    q = q3.reshape(B, K)
    p = p3.reshape(B, K)

    sr_pairs = jnp.concatenate(
        [srT, jnp.zeros((B, 1, C), jnp.float32)], axis=1).reshape(B * (K + 1), C)
    st_pairs = stT.reshape(B * (K + 1), C)
    u_col = uniform_sample.reshape(B * (K + 1), 1)

    cstar, base, thr, flag, na = _run_k2(sr_pairs, st_pairs, u_col, q, p,
                                         uniform_accept)

    thr9 = thr.reshape(B, 1, K + 1)
    base9 = base.reshape(B, 1, K + 1)
    flag9 = flag.reshape(B, 1, K + 1)
    na9 = na.reshape(B, 1, 1)
    ids9 = draft_token_ids.reshape(B, 1, K)
    cstar_flat = cstar.reshape(B * (K + 1))

    return _run_k3(cstar_flat, draft_probs, target_probs,
                   thr9, base9, flag9, na9, ids9)
